# baseline (speedup 1.0000x reference)
import functools

import jax
import jax.numpy as jnp
from jax import lax
from jax.experimental import pallas as pl
from jax.experimental.pallas import tpu as pltpu
from jax.experimental.pallas import tpu_sc as plsc

DIM = 2048
E = 16
KTOP = 2
F = 512
S = 2
SH = 1024
N = 2048
NSH = S * (SH // F)
TILE = 128
NT = 48
R = NT * TILE
TMR = 128
NCH = N // TMR
NW = 32
TPW = N // NW
APW = KTOP * N // NW


def _route_body(x_ref, rw_ref, rb_ref, xb_ref, eid_ref, wa_ref, wb_ref,
                cnt_out_ref, cnt_ref):
    t = pl.program_id(0)
    x = x_ref[...]
    xb_ref[...] = x.astype(jnp.bfloat16)
    logits = lax.dot_general(x, rw_ref[...], (((1,), (1,)), ((), ())),
                             preferred_element_type=jnp.float32)
    m = jnp.max(logits, axis=-1, keepdims=True)
    ex = jnp.exp(logits - m)
    scores = ex / jnp.sum(ex, axis=-1, keepdims=True)
    biased = logits + rb_ref[...]
    ids = lax.broadcasted_iota(jnp.int32, logits.shape, 1)
    i1 = jnp.argmax(biased, axis=-1)
    oh1 = (ids == i1[:, None])
    i2 = jnp.argmax(jnp.where(oh1, -jnp.inf, biased), axis=-1)
    oh2 = (ids == i2[:, None])
    s1 = jnp.sum(jnp.where(oh1, scores, 0.0), axis=-1, keepdims=True)
    s2 = jnp.sum(jnp.where(oh2, scores, 0.0), axis=-1, keepdims=True)
    tot = s1 + s2
    eid_ref[...] = jnp.concatenate([i1[:, None], i2[:, None]],
                                   axis=1).astype(jnp.int32)
    wa_ref[...] = s1 / tot
    wb_ref[...] = s2 / tot

    @pl.when(t == 0)
    def _():
        cnt_ref[...] = jnp.zeros((8, 128), jnp.float32)

    ohc = (jnp.concatenate([oh1, oh2], axis=0)).astype(jnp.float32)
    cnt_ref[0:1, 0:E] += jnp.sum(ohc, axis=0, keepdims=True)

    @pl.when(t == NCH - 1)
    def _():
        cnt_out_ref[...] = cnt_ref[0:1, 0:E]


def _plan_body(eid_ref, cin_ref, posa_ref, posb_ref, te_ref, tv_ref,
               chg_ref, fs_ref, scr_ref):
    t = pl.program_id(0)

    @pl.when(t == 0)
    def _():
        counts = cin_ref[...]
        tiles = jnp.floor((counts + (TILE - 1)) * (1.0 / TILE))
        lt = (lax.broadcasted_iota(jnp.int32, (E, E), 0)
              < lax.broadcasted_iota(jnp.int32, (E, E), 1)).astype(jnp.float32)
        offrows = lax.dot_general(tiles * TILE, lt, (((1,), (0,)), ((), ())),
                                  preferred_element_type=jnp.float32)
        tstart = lax.dot_general(tiles, lt, (((1,), (0,)), ((), ())),
                                 preferred_element_type=jnp.float32)
        total = jnp.sum(tiles, axis=1, keepdims=True)
        scr_ref[1:2, 0:E] = offrows
        scr_ref[0:1, 0:E] = jnp.zeros((1, E), jnp.float32)
        trow = lax.broadcasted_iota(jnp.int32, (NT, E), 0).astype(jnp.float32)
        ttc = jnp.minimum(trow, total - 1.0)
        owner = jnp.sum((tstart <= ttc).astype(jnp.float32), axis=1,
                        keepdims=True) - 1.0
        te_ref[...] = owner.astype(jnp.int32)
        tv_ref[...] = (lax.broadcasted_iota(jnp.int32, (NT, 1), 0)
                       .astype(jnp.float32) < total).astype(jnp.int32)
        prev = jnp.concatenate([owner[0:1] - 1.0, owner[:-1]], axis=0)
        chgf = (owner != prev).astype(jnp.float32)
        tln = (lax.broadcasted_iota(jnp.int32, (NT, NT), 0)
               >= lax.broadcasted_iota(jnp.int32, (NT, NT), 1)
               ).astype(jnp.float32)
        csum = lax.dot_general(tln, chgf, (((1,), (0,)), ((), ())),
                               preferred_element_type=jnp.float32)
        fsf = csum - 1.0
        chg_ref[...] = chgf.astype(jnp.int32)
        fs_ref[...] = (fsf - 2.0 * jnp.floor(fsf * 0.5)).astype(jnp.int32)

    eids = eid_ref[...]
    ids = lax.broadcasted_iota(jnp.int32, (TMR, E), 1)
    oh1 = (ids == eids[:, 0:1]).astype(jnp.float32)
    oh2 = (ids == eids[:, 1:2]).astype(jnp.float32)
    ohc = jnp.concatenate([oh1, oh2], axis=0)
    runb = scr_ref[0:1, 0:E]
    offr = scr_ref[1:2, 0:E]
    ltr = (lax.broadcasted_iota(jnp.int32, (2 * TMR, 2 * TMR), 0)
           > lax.broadcasted_iota(jnp.int32, (2 * TMR, 2 * TMR), 1)
           ).astype(jnp.float32)
    rkmat = lax.dot_general(ltr, ohc, (((1,), (0,)), ((), ())),
                            preferred_element_type=jnp.float32)
    rank = jnp.sum(rkmat * ohc, axis=1, keepdims=True)
    base = jnp.sum(ohc * (runb + offr), axis=1, keepdims=True)
    posv = (base + rank).astype(jnp.int32)
    posa_ref[...] = posv[:TMR]
    posb_ref[...] = posv[TMR:]
    scr_ref[0:1, 0:E] = runb + jnp.sum(ohc, axis=0, keepdims=True)


def _ffn_body(te_ref, tv_ref, chg_ref, fs_ref, xs_ref, gw_any, uw_any,
              dw_any, ws_ref, out_ref, gbuf, ubuf, dbuf, gsem, usem, dsem):
    t = pl.program_id(0)
    slot = fs_ref[t]

    @pl.when(t == 0)
    def _():
        e0 = te_ref[0]
        pltpu.make_async_copy(gw_any.at[e0], gbuf.at[slot], gsem).start()
        pltpu.make_async_copy(uw_any.at[e0], ubuf.at[slot], usem).start()
        pltpu.make_async_copy(dw_any.at[e0], dbuf.at[slot], dsem).start()

    @pl.when(chg_ref[t] == 1)
    def _():
        e = te_ref[t]
        pltpu.make_async_copy(gw_any.at[e], gbuf.at[slot], gsem).wait()
        pltpu.make_async_copy(uw_any.at[e], ubuf.at[slot], usem).wait()
        pltpu.make_async_copy(dw_any.at[e], dbuf.at[slot], dsem).wait()

    tn = jnp.minimum(t + 1, NT - 1)

    @pl.when(jnp.logical_and(t < NT - 1, chg_ref[tn] == 1))
    def _():
        en = te_ref[tn]
        ns = fs_ref[tn]
        pltpu.make_async_copy(gw_any.at[en], gbuf.at[ns], gsem).start()
        pltpu.make_async_copy(uw_any.at[en], ubuf.at[ns], usem).start()
        pltpu.make_async_copy(dw_any.at[en], dbuf.at[ns], dsem).start()

    @pl.when(tv_ref[t] == 1)
    def _():
        xs = xs_ref[...].astype(jnp.bfloat16)
        gw = gbuf[pl.ds(slot, 1)][0].astype(jnp.bfloat16)
        uw = ubuf[pl.ds(slot, 1)][0].astype(jnp.bfloat16)
        dw = dbuf[pl.ds(slot, 1)][0].astype(jnp.bfloat16)
        hg = lax.dot_general(xs, gw, (((1,), (1,)), ((), ())),
                             preferred_element_type=jnp.float32)
        hu = lax.dot_general(xs, uw, (((1,), (1,)), ((), ())),
                             preferred_element_type=jnp.float32)
        wcol = jnp.transpose(ws_ref[0], (1, 0))
        h = (hg * lax.logistic(hg)) * hu * wcol
        out_ref[...] = lax.dot_general(
            h.astype(jnp.bfloat16), dw, (((1,), (1,)), ((), ())),
            preferred_element_type=jnp.float32)


def _shared_body(xb_ref, gw_ref, uw_ref, dw_ref, out_ref):
    j = pl.program_id(1)
    xb = xb_ref[...]
    gw = gw_ref[0].astype(jnp.bfloat16)
    uw = uw_ref[0].astype(jnp.bfloat16)
    dw = dw_ref[0].astype(jnp.bfloat16)
    hg = lax.dot_general(xb, gw, (((1,), (1,)), ((), ())),
                         preferred_element_type=jnp.float32)
    hu = lax.dot_general(xb, uw, (((1,), (1,)), ((), ())),
                         preferred_element_type=jnp.float32)
    h = (hg * lax.logistic(hg)) * hu
    contrib = lax.dot_general(h.astype(jnp.bfloat16), dw,
                              (((1,), (1,)), ((), ())),
                              preferred_element_type=jnp.float32)

    @pl.when(j == 0)
    def _():
        out_ref[...] = contrib

    @pl.when(j > 0)
    def _():
        out_ref[...] += contrib


def _final_body(base_ref, ga0_ref, ga1_ref, out_ref):
    out_ref[...] = base_ref[...] + ga0_ref[...] + ga1_ref[...]


DCH = 32


def _build_dispatch():
    @functools.partial(
        pl.kernel,
        out_type=[
            jax.ShapeDtypeStruct((R, DIM), jnp.float32),
            jax.ShapeDtypeStruct((R,), jnp.float32),
        ],
        mesh=plsc.VectorSubcoreMesh(core_axis_name="c", subcore_axis_name="s"),
        scratch_types=[
            pltpu.VMEM((DCH, DIM), jnp.float32),
            pltpu.VMEM((DCH,), jnp.int32),
            pltpu.VMEM((DCH,), jnp.int32),
            pltpu.VMEM((DCH,), jnp.float32),
            pltpu.SemaphoreType.DMA,
            pltpu.SemaphoreType.DMA,
            pltpu.SemaphoreType.DMA,
        ],
    )
    def _dispatch(xf, posa, posb, wa, wb, sx, wsort, xrows, idxa, idxb, wbuf,
                  sema, semb, semw):
        wid = lax.axis_index("s") * 2 + lax.axis_index("c")
        for c in range(TPW // DCH):
            tbase = wid * TPW + c * DCH
            pltpu.sync_copy(xf.at[pl.ds(tbase, DCH)], xrows)
            pltpu.sync_copy(posa.at[pl.ds(tbase, DCH)], idxa)
            pltpu.sync_copy(posb.at[pl.ds(tbase, DCH)], idxb)
            cpa = pltpu.async_copy(xrows, sx.at[idxa], sema)
            cpb = pltpu.async_copy(xrows, sx.at[idxb], semb)
            pltpu.sync_copy(wa.at[pl.ds(tbase, DCH)], wbuf)
            pltpu.async_copy(wbuf, wsort.at[idxa], semw).wait()
            pltpu.sync_copy(wb.at[pl.ds(tbase, DCH)], wbuf)
            pltpu.async_copy(wbuf, wsort.at[idxb], semw).wait()
            cpa.wait()
            cpb.wait()

    return _dispatch


def _build_combine():
    @functools.partial(
        pl.kernel,
        out_type=[jax.ShapeDtypeStruct((KTOP * N, DIM), jnp.float32)],
        mesh=plsc.VectorSubcoreMesh(core_axis_name="c", subcore_axis_name="s"),
        scratch_types=[
            pltpu.VMEM((DCH, DIM), jnp.float32),
            pltpu.VMEM((DCH,), jnp.int32),
            pltpu.SemaphoreType.DMA,
        ],
    )
    def _combine(part, posa, posb, ga, rows, idx, sem):
        wid = lax.axis_index("s") * 2 + lax.axis_index("c")
        for k, posk in ((0, posa), (1, posb)):
            for c in range(TPW // DCH):
                tbase = wid * TPW + c * DCH
                pltpu.sync_copy(posk.at[pl.ds(tbase, DCH)], idx)
                pltpu.async_copy(part.at[idx], rows, sem).wait()
                pltpu.sync_copy(rows, ga.at[pl.ds(k * N + tbase, DCH)])

    return _combine


def kernel(x, router_w, router_bias, gate_w, up_w, down_w, sg_w, su_w, sd_w):
    B, T, D = x.shape
    flat = x.reshape(B * T, D)

    xb, eid, wa, wb, cnts = pl.pallas_call(
        _route_body,
        grid=(NCH,),
        in_specs=[
            pl.BlockSpec((TMR, DIM), lambda t: (t, 0)),
            pl.BlockSpec((E, DIM), lambda t: (0, 0)),
            pl.BlockSpec((1, E), lambda t: (0, 0)),
        ],
        out_specs=[
            pl.BlockSpec((TMR, DIM), lambda t: (t, 0)),
            pl.BlockSpec((TMR, KTOP), lambda t: (t, 0)),
            pl.BlockSpec((TMR, 1), lambda t: (t, 0)),
            pl.BlockSpec((TMR, 1), lambda t: (t, 0)),
            pl.BlockSpec((1, E), lambda t: (0, 0)),
        ],
        out_shape=[
            jax.ShapeDtypeStruct((N, DIM), jnp.bfloat16),
            jax.ShapeDtypeStruct((N, KTOP), jnp.int32),
            jax.ShapeDtypeStruct((N, 1), jnp.float32),
            jax.ShapeDtypeStruct((N, 1), jnp.float32),
            jax.ShapeDtypeStruct((1, E), jnp.float32),
        ],
        scratch_shapes=[pltpu.VMEM((8, 128), jnp.float32)],
        compiler_params=pltpu.CompilerParams(
            dimension_semantics=("arbitrary",)),
    )(flat, router_w, router_bias.reshape(1, E))

    posa2, posb2, te2, tv2, chg2, fs2 = pl.pallas_call(
        _plan_body,
        grid=(NCH,),
        in_specs=[
            pl.BlockSpec((TMR, KTOP), lambda t: (t, 0)),
            pl.BlockSpec((1, E), lambda t: (0, 0)),
        ],
        out_specs=[
            pl.BlockSpec((TMR, 1), lambda t: (t, 0)),
            pl.BlockSpec((TMR, 1), lambda t: (t, 0)),
            pl.BlockSpec((NT, 1), lambda t: (0, 0)),
            pl.BlockSpec((NT, 1), lambda t: (0, 0)),
            pl.BlockSpec((NT, 1), lambda t: (0, 0)),
            pl.BlockSpec((NT, 1), lambda t: (0, 0)),
        ],
        out_shape=[
            jax.ShapeDtypeStruct((N, 1), jnp.int32),
            jax.ShapeDtypeStruct((N, 1), jnp.int32),
            jax.ShapeDtypeStruct((NT, 1), jnp.int32),
            jax.ShapeDtypeStruct((NT, 1), jnp.int32),
            jax.ShapeDtypeStruct((NT, 1), jnp.int32),
            jax.ShapeDtypeStruct((NT, 1), jnp.int32),
        ],
        scratch_shapes=[pltpu.VMEM((8, 128), jnp.float32)],
        compiler_params=pltpu.CompilerParams(
            dimension_semantics=("arbitrary",)),
    )(eid, cnts)

    posa = posa2.reshape(N)
    posb = posb2.reshape(N)

    sortedx, wsort = _build_dispatch()(flat, posa, posb,
                                       wa.reshape(N), wb.reshape(N))
    ws3 = wsort.reshape(NT, 1, TILE)
    te = te2.reshape(NT)
    tv = tv2.reshape(NT)

    partial = pl.pallas_call(
        _ffn_body,
        grid_spec=pltpu.PrefetchScalarGridSpec(
            num_scalar_prefetch=4,
            grid=(NT,),
            in_specs=[
                pl.BlockSpec((TILE, DIM), lambda t, te, tv, ch, fs: (t, 0)),
                pl.BlockSpec(memory_space=pl.ANY),
                pl.BlockSpec(memory_space=pl.ANY),
                pl.BlockSpec(memory_space=pl.ANY),
                pl.BlockSpec((1, 1, TILE), lambda t, te, tv, ch, fs: (t, 0, 0)),
            ],
            out_specs=pl.BlockSpec((TILE, DIM), lambda t, te, tv, ch, fs: (t, 0)),
            scratch_shapes=[
                pltpu.VMEM((2, F, DIM), jnp.float32),
                pltpu.VMEM((2, F, DIM), jnp.float32),
                pltpu.VMEM((2, DIM, F), jnp.float32),
                pltpu.SemaphoreType.DMA,
                pltpu.SemaphoreType.DMA,
                pltpu.SemaphoreType.DMA,
            ],
        ),
        out_shape=jax.ShapeDtypeStruct((R, DIM), jnp.float32),
        compiler_params=pltpu.CompilerParams(
            dimension_semantics=("arbitrary",)),
    )(te, tv, chg2.reshape(NT), fs2.reshape(NT), sortedx,
      gate_w, up_w, down_w, ws3)

    ga, = _build_combine()(partial, posa, posb)

    TM2 = 1024
    base = pl.pallas_call(
        _shared_body,
        grid=(N // TM2, NSH),
        in_specs=[
            pl.BlockSpec((TM2, DIM), lambda i, j: (i, 0)),
            pl.BlockSpec((1, F, DIM), lambda i, j: (j // 2, j % 2, 0)),
            pl.BlockSpec((1, F, DIM), lambda i, j: (j // 2, j % 2, 0)),
            pl.BlockSpec((1, DIM, F), lambda i, j: (j // 2, 0, j % 2)),
        ],
        out_specs=pl.BlockSpec((TM2, DIM), lambda i, j: (i, 0)),
        out_shape=jax.ShapeDtypeStruct((N, DIM), jnp.float32),
        compiler_params=pltpu.CompilerParams(
            dimension_semantics=("arbitrary", "arbitrary")),
    )(xb, sg_w, su_w, sd_w)

    TM6 = 256
    nb6 = N // TM6
    out = pl.pallas_call(
        _final_body,
        grid=(nb6,),
        in_specs=[
            pl.BlockSpec((TM6, DIM), lambda i: (i, 0)),
            pl.BlockSpec((TM6, DIM), lambda i: (i, 0)),
            pl.BlockSpec((TM6, DIM), lambda i: (nb6 + i, 0)),
        ],
        out_specs=pl.BlockSpec((TM6, DIM), lambda i: (i, 0)),
        out_shape=jax.ShapeDtypeStruct((N, DIM), jnp.float32),
        compiler_params=pltpu.CompilerParams(
            dimension_semantics=("arbitrary",)),
    )(base, ga, ga)

    return out.reshape(B, T, D)


# speedup vs baseline: 1.0076x; 1.0076x over previous
import functools

import jax
import jax.numpy as jnp
from jax import lax
from jax.experimental import pallas as pl
from jax.experimental.pallas import tpu as pltpu
from jax.experimental.pallas import tpu_sc as plsc

DIM = 2048
E = 16
KTOP = 2
F = 512
S = 2
SH = 1024
N = 2048
NSH = S * (SH // F)
TILE = 128
NT = 48
R = NT * TILE
TMR = 128
NCH = N // TMR
NW = 32
TPW = N // NW
APW = KTOP * N // NW


def _route_body(x_ref, rw_ref, rb_ref, xb_ref, eid_ref, wa_ref, wb_ref,
                cnt_out_ref, cnt_ref):
    t = pl.program_id(0)
    x = x_ref[...]
    xb_ref[...] = x.astype(jnp.bfloat16)
    logits = lax.dot_general(x, rw_ref[...], (((1,), (1,)), ((), ())),
                             preferred_element_type=jnp.float32)
    m = jnp.max(logits, axis=-1, keepdims=True)
    ex = jnp.exp(logits - m)
    scores = ex / jnp.sum(ex, axis=-1, keepdims=True)
    biased = logits + rb_ref[...]
    ids = lax.broadcasted_iota(jnp.int32, logits.shape, 1)
    i1 = jnp.argmax(biased, axis=-1)
    oh1 = (ids == i1[:, None])
    i2 = jnp.argmax(jnp.where(oh1, -jnp.inf, biased), axis=-1)
    oh2 = (ids == i2[:, None])
    s1 = jnp.sum(jnp.where(oh1, scores, 0.0), axis=-1, keepdims=True)
    s2 = jnp.sum(jnp.where(oh2, scores, 0.0), axis=-1, keepdims=True)
    tot = s1 + s2
    eid_ref[...] = jnp.concatenate([i1[:, None], i2[:, None]],
                                   axis=1).astype(jnp.int32)
    wa_ref[...] = s1 / tot
    wb_ref[...] = s2 / tot

    @pl.when(t == 0)
    def _():
        cnt_ref[...] = jnp.zeros((8, 128), jnp.float32)

    ohc = (jnp.concatenate([oh1, oh2], axis=0)).astype(jnp.float32)
    cnt_ref[0:1, 0:E] += jnp.sum(ohc, axis=0, keepdims=True)

    @pl.when(t == NCH - 1)
    def _():
        cnt_out_ref[...] = cnt_ref[0:1, 0:E]


def _plan_body(eid_ref, cin_ref, posa_ref, posb_ref, te_ref, tv_ref,
               chg_ref, fs_ref, scr_ref):
    t = pl.program_id(0)

    @pl.when(t == 0)
    def _():
        counts = cin_ref[...]
        tiles = jnp.floor((counts + (TILE - 1)) * (1.0 / TILE))
        lt = (lax.broadcasted_iota(jnp.int32, (E, E), 0)
              < lax.broadcasted_iota(jnp.int32, (E, E), 1)).astype(jnp.float32)
        offrows = lax.dot_general(tiles * TILE, lt, (((1,), (0,)), ((), ())),
                                  preferred_element_type=jnp.float32)
        tstart = lax.dot_general(tiles, lt, (((1,), (0,)), ((), ())),
                                 preferred_element_type=jnp.float32)
        total = jnp.sum(tiles, axis=1, keepdims=True)
        scr_ref[1:2, 0:E] = offrows
        scr_ref[0:1, 0:E] = jnp.zeros((1, E), jnp.float32)
        trow = lax.broadcasted_iota(jnp.int32, (NT, E), 0).astype(jnp.float32)
        ttc = jnp.minimum(trow, total - 1.0)
        owner = jnp.sum((tstart <= ttc).astype(jnp.float32), axis=1,
                        keepdims=True) - 1.0
        te_ref[...] = owner.astype(jnp.int32)
        tv_ref[...] = (lax.broadcasted_iota(jnp.int32, (NT, 1), 0)
                       .astype(jnp.float32) < total).astype(jnp.int32)
        prev = jnp.concatenate([owner[0:1] - 1.0, owner[:-1]], axis=0)
        chgf = (owner != prev).astype(jnp.float32)
        tln = (lax.broadcasted_iota(jnp.int32, (NT, NT), 0)
               >= lax.broadcasted_iota(jnp.int32, (NT, NT), 1)
               ).astype(jnp.float32)
        csum = lax.dot_general(tln, chgf, (((1,), (0,)), ((), ())),
                               preferred_element_type=jnp.float32)
        fsf = csum - 1.0
        chg_ref[...] = chgf.astype(jnp.int32)
        fs_ref[...] = (fsf - 2.0 * jnp.floor(fsf * 0.5)).astype(jnp.int32)

    eids = eid_ref[...]
    ids = lax.broadcasted_iota(jnp.int32, (TMR, E), 1)
    oh1 = (ids == eids[:, 0:1]).astype(jnp.float32)
    oh2 = (ids == eids[:, 1:2]).astype(jnp.float32)
    ohc = jnp.concatenate([oh1, oh2], axis=0)
    runb = scr_ref[0:1, 0:E]
    offr = scr_ref[1:2, 0:E]
    ltr = (lax.broadcasted_iota(jnp.int32, (2 * TMR, 2 * TMR), 0)
           > lax.broadcasted_iota(jnp.int32, (2 * TMR, 2 * TMR), 1)
           ).astype(jnp.float32)
    rkmat = lax.dot_general(ltr, ohc, (((1,), (0,)), ((), ())),
                            preferred_element_type=jnp.float32)
    rank = jnp.sum(rkmat * ohc, axis=1, keepdims=True)
    base = jnp.sum(ohc * (runb + offr), axis=1, keepdims=True)
    posv = (base + rank).astype(jnp.int32)
    posa_ref[...] = posv[:TMR]
    posb_ref[...] = posv[TMR:]
    scr_ref[0:1, 0:E] = runb + jnp.sum(ohc, axis=0, keepdims=True)


def _ffn_body(te_ref, tv_ref, chg_ref, fs_ref, xs_ref, gw_any, uw_any,
              dw_any, ws_ref, out_ref, gbuf, ubuf, dbuf, gsem, usem, dsem):
    t = pl.program_id(0)
    slot = fs_ref[t]

    @pl.when(t == 0)
    def _():
        e0 = te_ref[0]
        pltpu.make_async_copy(gw_any.at[e0], gbuf.at[slot], gsem).start()
        pltpu.make_async_copy(uw_any.at[e0], ubuf.at[slot], usem).start()
        pltpu.make_async_copy(dw_any.at[e0], dbuf.at[slot], dsem).start()

    @pl.when(chg_ref[t] == 1)
    def _():
        e = te_ref[t]
        pltpu.make_async_copy(gw_any.at[e], gbuf.at[slot], gsem).wait()
        pltpu.make_async_copy(uw_any.at[e], ubuf.at[slot], usem).wait()
        pltpu.make_async_copy(dw_any.at[e], dbuf.at[slot], dsem).wait()

    tn = jnp.minimum(t + 1, NT - 1)

    @pl.when(jnp.logical_and(t < NT - 1, chg_ref[tn] == 1))
    def _():
        en = te_ref[tn]
        ns = fs_ref[tn]
        pltpu.make_async_copy(gw_any.at[en], gbuf.at[ns], gsem).start()
        pltpu.make_async_copy(uw_any.at[en], ubuf.at[ns], usem).start()
        pltpu.make_async_copy(dw_any.at[en], dbuf.at[ns], dsem).start()

    @pl.when(tv_ref[t] == 1)
    def _():
        xs = xs_ref[...]
        gw = gbuf[pl.ds(slot, 1)][0]
        uw = ubuf[pl.ds(slot, 1)][0]
        dw = dbuf[pl.ds(slot, 1)][0]
        hg = lax.dot_general(xs, gw, (((1,), (1,)), ((), ())),
                             preferred_element_type=jnp.float32)
        hu = lax.dot_general(xs, uw, (((1,), (1,)), ((), ())),
                             preferred_element_type=jnp.float32)
        wcol = jnp.transpose(ws_ref[0], (1, 0))
        h = (hg * lax.logistic(hg)) * hu * wcol
        out_ref[...] = lax.dot_general(h, dw, (((1,), (1,)), ((), ())),
                                       preferred_element_type=jnp.float32)


def _shared_body(xb_ref, gw_ref, uw_ref, dw_ref, out_ref):
    j = pl.program_id(1)
    xb = xb_ref[...]
    gw = gw_ref[0].astype(jnp.bfloat16)
    uw = uw_ref[0].astype(jnp.bfloat16)
    dw = dw_ref[0].astype(jnp.bfloat16)
    hg = lax.dot_general(xb, gw, (((1,), (1,)), ((), ())),
                         preferred_element_type=jnp.float32)
    hu = lax.dot_general(xb, uw, (((1,), (1,)), ((), ())),
                         preferred_element_type=jnp.float32)
    h = (hg * lax.logistic(hg)) * hu
    contrib = lax.dot_general(h.astype(jnp.bfloat16), dw,
                              (((1,), (1,)), ((), ())),
                              preferred_element_type=jnp.float32)

    @pl.when(j == 0)
    def _():
        out_ref[...] = contrib

    @pl.when(j > 0)
    def _():
        out_ref[...] += contrib


def _final_body(base_ref, ga0_ref, ga1_ref, out_ref):
    out_ref[...] = base_ref[...] + ga0_ref[...] + ga1_ref[...]


DCH = 32


def _build_dispatch():
    @functools.partial(
        pl.kernel,
        out_type=[
            jax.ShapeDtypeStruct((R, DIM), jnp.float32),
            jax.ShapeDtypeStruct((R,), jnp.float32),
        ],
        mesh=plsc.VectorSubcoreMesh(core_axis_name="c", subcore_axis_name="s"),
        scratch_types=[
            pltpu.VMEM((DCH, DIM), jnp.float32),
            pltpu.VMEM((DCH,), jnp.int32),
            pltpu.VMEM((DCH,), jnp.int32),
            pltpu.VMEM((DCH,), jnp.float32),
            pltpu.SemaphoreType.DMA,
            pltpu.SemaphoreType.DMA,
            pltpu.SemaphoreType.DMA,
        ],
    )
    def _dispatch(xf, posa, posb, wa, wb, sx, wsort, xrows, idxa, idxb, wbuf,
                  sema, semb, semw):
        wid = lax.axis_index("s") * 2 + lax.axis_index("c")
        for c in range(TPW // DCH):
            tbase = wid * TPW + c * DCH
            pltpu.sync_copy(xf.at[pl.ds(tbase, DCH)], xrows)
            pltpu.sync_copy(posa.at[pl.ds(tbase, DCH)], idxa)
            pltpu.sync_copy(posb.at[pl.ds(tbase, DCH)], idxb)
            cpa = pltpu.async_copy(xrows, sx.at[idxa], sema)
            cpb = pltpu.async_copy(xrows, sx.at[idxb], semb)
            pltpu.sync_copy(wa.at[pl.ds(tbase, DCH)], wbuf)
            pltpu.async_copy(wbuf, wsort.at[idxa], semw).wait()
            pltpu.sync_copy(wb.at[pl.ds(tbase, DCH)], wbuf)
            pltpu.async_copy(wbuf, wsort.at[idxb], semw).wait()
            cpa.wait()
            cpb.wait()

    return _dispatch


def _build_combine():
    @functools.partial(
        pl.kernel,
        out_type=[jax.ShapeDtypeStruct((KTOP * N, DIM), jnp.float32)],
        mesh=plsc.VectorSubcoreMesh(core_axis_name="c", subcore_axis_name="s"),
        scratch_types=[
            pltpu.VMEM((DCH, DIM), jnp.float32),
            pltpu.VMEM((DCH,), jnp.int32),
            pltpu.SemaphoreType.DMA,
        ],
    )
    def _combine(part, posa, posb, ga, rows, idx, sem):
        wid = lax.axis_index("s") * 2 + lax.axis_index("c")
        for k, posk in ((0, posa), (1, posb)):
            for c in range(TPW // DCH):
                tbase = wid * TPW + c * DCH
                pltpu.sync_copy(posk.at[pl.ds(tbase, DCH)], idx)
                pltpu.async_copy(part.at[idx], rows, sem).wait()
                pltpu.sync_copy(rows, ga.at[pl.ds(k * N + tbase, DCH)])

    return _combine


def kernel(x, router_w, router_bias, gate_w, up_w, down_w, sg_w, su_w, sd_w):
    B, T, D = x.shape
    flat = x.reshape(B * T, D)

    xb, eid, wa, wb, cnts = pl.pallas_call(
        _route_body,
        grid=(NCH,),
        in_specs=[
            pl.BlockSpec((TMR, DIM), lambda t: (t, 0)),
            pl.BlockSpec((E, DIM), lambda t: (0, 0)),
            pl.BlockSpec((1, E), lambda t: (0, 0)),
        ],
        out_specs=[
            pl.BlockSpec((TMR, DIM), lambda t: (t, 0)),
            pl.BlockSpec((TMR, KTOP), lambda t: (t, 0)),
            pl.BlockSpec((TMR, 1), lambda t: (t, 0)),
            pl.BlockSpec((TMR, 1), lambda t: (t, 0)),
            pl.BlockSpec((1, E), lambda t: (0, 0)),
        ],
        out_shape=[
            jax.ShapeDtypeStruct((N, DIM), jnp.bfloat16),
            jax.ShapeDtypeStruct((N, KTOP), jnp.int32),
            jax.ShapeDtypeStruct((N, 1), jnp.float32),
            jax.ShapeDtypeStruct((N, 1), jnp.float32),
            jax.ShapeDtypeStruct((1, E), jnp.float32),
        ],
        scratch_shapes=[pltpu.VMEM((8, 128), jnp.float32)],
        compiler_params=pltpu.CompilerParams(
            dimension_semantics=("arbitrary",)),
    )(flat, router_w, router_bias.reshape(1, E))

    posa2, posb2, te2, tv2, chg2, fs2 = pl.pallas_call(
        _plan_body,
        grid=(NCH,),
        in_specs=[
            pl.BlockSpec((TMR, KTOP), lambda t: (t, 0)),
            pl.BlockSpec((1, E), lambda t: (0, 0)),
        ],
        out_specs=[
            pl.BlockSpec((TMR, 1), lambda t: (t, 0)),
            pl.BlockSpec((TMR, 1), lambda t: (t, 0)),
            pl.BlockSpec((NT, 1), lambda t: (0, 0)),
            pl.BlockSpec((NT, 1), lambda t: (0, 0)),
            pl.BlockSpec((NT, 1), lambda t: (0, 0)),
            pl.BlockSpec((NT, 1), lambda t: (0, 0)),
        ],
        out_shape=[
            jax.ShapeDtypeStruct((N, 1), jnp.int32),
            jax.ShapeDtypeStruct((N, 1), jnp.int32),
            jax.ShapeDtypeStruct((NT, 1), jnp.int32),
            jax.ShapeDtypeStruct((NT, 1), jnp.int32),
            jax.ShapeDtypeStruct((NT, 1), jnp.int32),
            jax.ShapeDtypeStruct((NT, 1), jnp.int32),
        ],
        scratch_shapes=[pltpu.VMEM((8, 128), jnp.float32)],
        compiler_params=pltpu.CompilerParams(
            dimension_semantics=("arbitrary",)),
    )(eid, cnts)

    posa = posa2.reshape(N)
    posb = posb2.reshape(N)

    sortedx, wsort = _build_dispatch()(flat, posa, posb,
                                       wa.reshape(N), wb.reshape(N))
    ws3 = wsort.reshape(NT, 1, TILE)
    te = te2.reshape(NT)
    tv = tv2.reshape(NT)

    partial = pl.pallas_call(
        _ffn_body,
        grid_spec=pltpu.PrefetchScalarGridSpec(
            num_scalar_prefetch=4,
            grid=(NT,),
            in_specs=[
                pl.BlockSpec((TILE, DIM), lambda t, te, tv, ch, fs: (t, 0)),
                pl.BlockSpec(memory_space=pl.ANY),
                pl.BlockSpec(memory_space=pl.ANY),
                pl.BlockSpec(memory_space=pl.ANY),
                pl.BlockSpec((1, 1, TILE), lambda t, te, tv, ch, fs: (t, 0, 0)),
            ],
            out_specs=pl.BlockSpec((TILE, DIM), lambda t, te, tv, ch, fs: (t, 0)),
            scratch_shapes=[
                pltpu.VMEM((2, F, DIM), jnp.float32),
                pltpu.VMEM((2, F, DIM), jnp.float32),
                pltpu.VMEM((2, DIM, F), jnp.float32),
                pltpu.SemaphoreType.DMA,
                pltpu.SemaphoreType.DMA,
                pltpu.SemaphoreType.DMA,
            ],
        ),
        out_shape=jax.ShapeDtypeStruct((R, DIM), jnp.float32),
        compiler_params=pltpu.CompilerParams(
            dimension_semantics=("arbitrary",)),
    )(te, tv, chg2.reshape(NT), fs2.reshape(NT), sortedx,
      gate_w, up_w, down_w, ws3)

    ga, = _build_combine()(partial, posa, posb)

    TM2 = 1024
    base = pl.pallas_call(
        _shared_body,
        grid=(N // TM2, NSH),
        in_specs=[
            pl.BlockSpec((TM2, DIM), lambda i, j: (i, 0)),
            pl.BlockSpec((1, F, DIM), lambda i, j: (j // 2, j % 2, 0)),
            pl.BlockSpec((1, F, DIM), lambda i, j: (j // 2, j % 2, 0)),
            pl.BlockSpec((1, DIM, F), lambda i, j: (j // 2, 0, j % 2)),
        ],
        out_specs=pl.BlockSpec((TM2, DIM), lambda i, j: (i, 0)),
        out_shape=jax.ShapeDtypeStruct((N, DIM), jnp.float32),
        compiler_params=pltpu.CompilerParams(
            dimension_semantics=("arbitrary", "arbitrary")),
    )(xb, sg_w, su_w, sd_w)

    TM6 = 256
    nb6 = N // TM6
    out = pl.pallas_call(
        _final_body,
        grid=(nb6,),
        in_specs=[
            pl.BlockSpec((TM6, DIM), lambda i: (i, 0)),
            pl.BlockSpec((TM6, DIM), lambda i: (i, 0)),
            pl.BlockSpec((TM6, DIM), lambda i: (nb6 + i, 0)),
        ],
        out_specs=pl.BlockSpec((TM6, DIM), lambda i: (i, 0)),
        out_shape=jax.ShapeDtypeStruct((N, DIM), jnp.float32),
        compiler_params=pltpu.CompilerParams(
            dimension_semantics=("arbitrary",)),
    )(base, ga, ga)

    return out.reshape(B, T, D)
